# Initial kernel scaffold; baseline (speedup 1.0000x reference)
import jax
import jax.numpy as jnp
from jax.experimental import pallas as pl


def kernel(V_set, E_set, a_node_ids, b_node_ids):
    raise NotImplementedError("write your pallas kernel here")


# speedup vs baseline: 1.5881x; 1.5881x over previous
import functools

import jax
import jax.numpy as jnp
from jax import lax
from jax.experimental import pallas as pl
from jax.experimental.pallas import tpu as pltpu
from jax.experimental.pallas import tpu_sc as plsc

N_NODES = 10000
N_EDGES = 320000
D_FEAT = 128
D_EDGE = 16
D_OUT = D_EDGE + 2 * D_FEAT

NUM_CORES = 2
NUM_SUBCORES = 16
NW = NUM_CORES * NUM_SUBCORES
E_PER_W = N_EDGES // NW
B = 80
G = B // 8
NCHUNK = E_PER_W // B


def _body(v_hbm, e_hbm, a_hbm, b_hbm, out_hbm, idx_a, idx_b,
          rows_a, rows_b, e_buf, buf, sem):
    wid = lax.axis_index("s") * NUM_CORES + lax.axis_index("c")
    base0 = wid * E_PER_W

    def chunk(c, carry):
        base = base0 + c * B
        pltpu.sync_copy(a_hbm.at[pl.ds(base, B)], idx_a)
        pltpu.sync_copy(b_hbm.at[pl.ds(base, B)], idx_b)
        cp_e = pltpu.async_copy(
            e_hbm.at[pl.ds(base, B), :], e_buf, sem)
        cp_a = pltpu.async_copy(v_hbm.at[idx_a], rows_a, sem)
        cp_b = pltpu.async_copy(v_hbm.at[idx_b], rows_b, sem)
        cp_e.wait()
        cp_a.wait()
        cp_b.wait()

        def group(g, carry2):
            for l in range(8):
                row = g * 8 + l
                buf[row, pl.ds(0, D_EDGE)] = e_buf[row, pl.ds(0, 16)]
                for j in range(8):
                    buf[row, pl.ds(D_EDGE + 16 * j, 16)] = (
                        rows_a[row, pl.ds(16 * j, 16)])
                    buf[row, pl.ds(D_EDGE + D_FEAT + 16 * j, 16)] = (
                        rows_b[row, pl.ds(16 * j, 16)])
            return carry2

        lax.fori_loop(0, G, group, 0)
        pltpu.sync_copy(buf, out_hbm.at[pl.ds(base, B), :])
        return carry

    lax.fori_loop(0, NCHUNK, chunk, 0)


@jax.jit
def _run(v, e2, ia, ib):
    mesh = plsc.VectorSubcoreMesh(core_axis_name="c", subcore_axis_name="s")
    kern = functools.partial(
        pl.kernel,
        mesh=mesh,
        out_type=jax.ShapeDtypeStruct((N_EDGES, D_OUT), jnp.float32),
        scratch_types=[
            pltpu.VMEM((B,), jnp.int32),
            pltpu.VMEM((B,), jnp.int32),
            pltpu.VMEM((B, D_FEAT), jnp.float32),
            pltpu.VMEM((B, D_FEAT), jnp.float32),
            pltpu.VMEM((B, D_EDGE), jnp.float32),
            pltpu.VMEM((B, D_OUT), jnp.float32),
            pltpu.SemaphoreType.DMA,
        ],
    )(_body)
    return kern(v, e2, ia, ib)


def kernel(V_set, E_set, a_node_ids, b_node_ids):
    v = V_set[0]
    e = E_set[0]
    ia = a_node_ids[0].astype(jnp.int32)
    ib = b_node_ids[0].astype(jnp.int32)
    out = _run(v, e, ia, ib)
    return out[jnp.newaxis, ...]


# speedup vs baseline: 5.3844x; 3.3904x over previous
import functools

import jax
import jax.numpy as jnp
from jax import lax
from jax.experimental import pallas as pl
from jax.experimental.pallas import tpu as pltpu
from jax.experimental.pallas import tpu_sc as plsc

N_NODES = 10000
N_EDGES = 320000
D_FEAT = 128
D_EDGE = 16
D_OUT = D_EDGE + 2 * D_FEAT

NUM_CORES = 2
NUM_SUBCORES = 16
NW = NUM_CORES * NUM_SUBCORES
C = 1280
NCH = N_EDGES // C
NPAIR = NCH // 2
W_TBL = 8 * N_NODES


def _body(vt_hbm, eT_hbm, a_hbm, b_hbm, outT_hbm, tbl, idx, ob0, ob1,
          sem0, sem1):
    wid = lax.axis_index("s") * NUM_CORES + lax.axis_index("c")
    t = wid // NUM_SUBCORES
    rb = wid % NUM_SUBCORES

    pltpu.sync_copy(vt_hbm.at[pl.ds(rb * W_TBL, W_TBL)], tbl)

    def e_item(j, carry):
        i = wid + NW * j

        @pl.when(i < 2 * NCH)
        def _():
            r = pl.multiple_of(jnp.where(i < NCH, 0, 8), 8)
            cc = jnp.where(i < NCH, i, i - NCH)
            col = pl.multiple_of(cc * C, 128)
            pltpu.sync_copy(eT_hbm.at[pl.ds(r, 8), pl.ds(col, C)], ob0)
            pltpu.sync_copy(ob0, outT_hbm.at[pl.ds(r, 8), pl.ds(col, C)])

        return carry

    lax.fori_loop(0, (2 * NCH + NW - 1) // NW, e_item, 0)

    row0 = D_EDGE + t * D_FEAT + rb * 8

    def v_pass(ids_hbm):
        def pair(t2, carry):
            for half, (ob, sem) in enumerate(((ob0, sem0), (ob1, sem1))):
                c = 2 * t2 + half
                col = pl.multiple_of(c * C, 128)

                @pl.when(t2 > 0)
                def _():
                    prev = pl.multiple_of((c - 2) * C, 128)
                    pltpu.make_async_copy(
                        ob, outT_hbm.at[pl.ds(row0, 8), pl.ds(prev, C)],
                        sem).wait()

                pltpu.sync_copy(ids_hbm.at[pl.ds(col, C)], idx)

                @plsc.parallel_loop(0, C // 16)
                def g_body(g):
                    iv = idx[pl.ds(g * 16, 16)]
                    for k in range(8):
                        vals = plsc.load_gather(
                            tbl.at[pl.ds(k * N_NODES, N_NODES)], [iv])
                        ob[k, pl.ds(g * 16, 16)] = vals

                pltpu.async_copy(
                    ob, outT_hbm.at[pl.ds(row0, 8), pl.ds(col, C)], sem)
            return carry

        lax.fori_loop(0, NPAIR, pair, 0)
        last = (NCH - 2) * C
        pltpu.make_async_copy(
            ob0, outT_hbm.at[pl.ds(row0, 8), pl.ds(last, C)], sem0).wait()
        pltpu.make_async_copy(
            ob1, outT_hbm.at[pl.ds(row0, 8), pl.ds(last + C, C)], sem1).wait()

    @pl.when(t == 0)
    def _():
        v_pass(a_hbm)

    @pl.when(t == 1)
    def _():
        v_pass(b_hbm)


@jax.jit
def _run(vt_flat, eT, ia, ib):
    mesh = plsc.VectorSubcoreMesh(core_axis_name="c", subcore_axis_name="s")
    kern = functools.partial(
        pl.kernel,
        mesh=mesh,
        out_type=jax.ShapeDtypeStruct((D_OUT, N_EDGES), jnp.float32),
        scratch_types=[
            pltpu.VMEM((NW * W_TBL // NW,), jnp.float32),
            pltpu.VMEM((C,), jnp.int32),
            pltpu.VMEM((8, C), jnp.float32),
            pltpu.VMEM((8, C), jnp.float32),
            pltpu.SemaphoreType.DMA,
            pltpu.SemaphoreType.DMA,
        ],
        compiler_params=pltpu.CompilerParams(needs_layout_passes=False),
    )(_body)
    return kern(vt_flat, eT, ia, ib)


def kernel(V_set, E_set, a_node_ids, b_node_ids):
    vt_flat = jnp.transpose(V_set[0]).reshape(-1)
    eT = jnp.transpose(E_set[0])
    ia = a_node_ids[0].astype(jnp.int32)
    ib = b_node_ids[0].astype(jnp.int32)
    outT = _run(vt_flat, eT, ia, ib)
    return jnp.transpose(outT)[jnp.newaxis, ...]


# speedup vs baseline: 7.9510x; 1.4767x over previous
import functools

import jax
import jax.numpy as jnp
from jax import lax
from jax.experimental import pallas as pl
from jax.experimental.pallas import tpu as pltpu
from jax.experimental.pallas import tpu_sc as plsc

N_NODES = 10000
N_EDGES = 320000
D_FEAT = 128
D_EDGE = 16
D_OUT = D_EDGE + 2 * D_FEAT

NUM_CORES = 2
NUM_SUBCORES = 16
NW = NUM_CORES * NUM_SUBCORES
C = 1280
NCH = N_EDGES // C
NPAIR = NCH // 2
W_TBL = 8 * N_NODES


def _body(vt_hbm, eT_hbm, a_hbm, b_hbm, outT_hbm, tbl, idx0, idx1, ob0, ob1,
          sem0, sem1, isem0, isem1):
    wid = lax.axis_index("s") * NUM_CORES + lax.axis_index("c")
    t = wid // NUM_SUBCORES
    rb = wid % NUM_SUBCORES

    pltpu.sync_copy(vt_hbm.at[pl.ds(rb * W_TBL, W_TBL)], tbl)

    def e_item(j, carry):
        i = wid + NW * j

        @pl.when(i < 2 * NCH)
        def _():
            r = pl.multiple_of(jnp.where(i < NCH, 0, 8), 8)
            cc = jnp.where(i < NCH, i, i - NCH)
            col = pl.multiple_of(cc * C, 128)
            pltpu.sync_copy(eT_hbm.at[pl.ds(r, 8), pl.ds(col, C)], ob0)
            pltpu.sync_copy(ob0, outT_hbm.at[pl.ds(r, 8), pl.ds(col, C)])

        return carry

    lax.fori_loop(0, (2 * NCH + NW - 1) // NW, e_item, 0)

    row0 = D_EDGE + t * D_FEAT + rb * 8

    def v_pass(ids_hbm):
        pltpu.async_copy(ids_hbm.at[pl.ds(0, C)], idx0, isem0)
        pltpu.async_copy(ids_hbm.at[pl.ds(C, C)], idx1, isem1)

        def pair(t2, carry):
            for half, (ob, sem, ixb, isem) in enumerate(
                    ((ob0, sem0, idx0, isem0), (ob1, sem1, idx1, isem1))):
                c = 2 * t2 + half
                col = pl.multiple_of(c * C, 128)
                pltpu.make_async_copy(
                    ids_hbm.at[pl.ds(col, C)], ixb, isem).wait()

                @pl.when(t2 > 0)
                def _():
                    prev = pl.multiple_of((c - 2) * C, 128)
                    pltpu.make_async_copy(
                        ob, outT_hbm.at[pl.ds(row0, 8), pl.ds(prev, C)],
                        sem).wait()

                @plsc.parallel_loop(0, C // 16)
                def g_body(g):
                    iv = ixb[pl.ds(g * 16, 16)]
                    for k in range(8):
                        vals = plsc.load_gather(
                            tbl.at[pl.ds(k * N_NODES, N_NODES)], [iv])
                        ob[k, pl.ds(g * 16, 16)] = vals

                @pl.when(c + 2 < NCH)
                def _():
                    nxt = pl.multiple_of((c + 2) * C, 128)
                    pltpu.async_copy(ids_hbm.at[pl.ds(nxt, C)], ixb, isem)

                pltpu.async_copy(
                    ob, outT_hbm.at[pl.ds(row0, 8), pl.ds(col, C)], sem)
            return carry

        lax.fori_loop(0, NPAIR, pair, 0)
        last = (NCH - 2) * C
        pltpu.make_async_copy(
            ob0, outT_hbm.at[pl.ds(row0, 8), pl.ds(last, C)], sem0).wait()
        pltpu.make_async_copy(
            ob1, outT_hbm.at[pl.ds(row0, 8), pl.ds(last + C, C)], sem1).wait()

    @pl.when(t == 0)
    def _():
        v_pass(a_hbm)

    @pl.when(t == 1)
    def _():
        v_pass(b_hbm)


@jax.jit
def _run(vt_flat, eT, ia, ib):
    mesh = plsc.VectorSubcoreMesh(core_axis_name="c", subcore_axis_name="s")
    kern = functools.partial(
        pl.kernel,
        mesh=mesh,
        out_type=jax.ShapeDtypeStruct((D_OUT, N_EDGES), jnp.float32),
        scratch_types=[
            pltpu.VMEM((W_TBL,), jnp.float32),
            pltpu.VMEM((C,), jnp.int32),
            pltpu.VMEM((C,), jnp.int32),
            pltpu.VMEM((8, C), jnp.float32),
            pltpu.VMEM((8, C), jnp.float32),
            pltpu.SemaphoreType.DMA,
            pltpu.SemaphoreType.DMA,
            pltpu.SemaphoreType.DMA,
            pltpu.SemaphoreType.DMA,
        ],
        compiler_params=pltpu.CompilerParams(needs_layout_passes=False),
    )(_body)
    return kern(vt_flat, eT, ia, ib)


def kernel(V_set, E_set, a_node_ids, b_node_ids):
    vt_flat = jnp.transpose(V_set[0]).reshape(-1)
    eT = jnp.transpose(E_set[0])
    ia = a_node_ids[0].astype(jnp.int32)
    ib = b_node_ids[0].astype(jnp.int32)
    outT = _run(vt_flat, eT, ia, ib)
    return jnp.transpose(outT)[jnp.newaxis, ...]


# speedup vs baseline: 8.5778x; 1.0788x over previous
import functools

import jax
import jax.numpy as jnp
from jax import lax
from jax.experimental import pallas as pl
from jax.experimental.pallas import tpu as pltpu
from jax.experimental.pallas import tpu_sc as plsc

N_NODES = 10000
N_EDGES = 320000
D_FEAT = 128
D_EDGE = 16
D_OUT = D_EDGE + 2 * D_FEAT

NUM_CORES = 2
NUM_SUBCORES = 16
NW = NUM_CORES * NUM_SUBCORES
C = 1280
NCH = N_EDGES // C
NPAIR = NCH // 2
W_TBL = 8 * N_NODES


def _body(vt_hbm, eT_hbm, a_hbm, b_hbm, outT_hbm, tbl, idx0, idx1, ob0, ob1,
          eb, sem0, sem1, isem0, isem1, tsem, ersem, ewsem):
    wid = lax.axis_index("s") * NUM_CORES + lax.axis_index("c")
    t = wid // NUM_SUBCORES
    rb = wid % NUM_SUBCORES

    pltpu.async_copy(vt_hbm.at[pl.ds(rb * W_TBL, W_TBL)], tbl, tsem)

    NE = 2 * NCH
    NE_IT = (NE + NW - 1) // NW

    def e_refs(i):
        r = pl.multiple_of(jnp.where(i < NCH, 0, 8), 8)
        cc = jnp.where(i < NCH, i, i - NCH)
        col = pl.multiple_of(cc * C, 128)
        return (eT_hbm.at[pl.ds(r, 8), pl.ds(col, C)],
                outT_hbm.at[pl.ds(r, 8), pl.ds(col, C)])

    row0 = D_EDGE + t * D_FEAT + rb * 8

    def v_pass(ids_hbm):
        pltpu.async_copy(ids_hbm.at[pl.ds(0, C)], idx0, isem0)
        pltpu.async_copy(ids_hbm.at[pl.ds(C, C)], idx1, isem1)
        pltpu.make_async_copy(
            vt_hbm.at[pl.ds(rb * W_TBL, W_TBL)], tbl, tsem).wait()

        def pair(t2, carry):
            j = t2 // 2
            i = wid + NW * j

            @pl.when(jnp.logical_and(t2 % 2 == 0, i < NE))
            def _():
                @pl.when(j > 0)
                def _():
                    _, dst_prev = e_refs(i - NW)
                    pltpu.make_async_copy(eb, dst_prev, ewsem).wait()

                src, _ = e_refs(i)
                pltpu.async_copy(src, eb, ersem)

            @pl.when(jnp.logical_and(t2 % 2 == 1, i < NE))
            def _():
                src, dst = e_refs(i)
                pltpu.make_async_copy(src, eb, ersem).wait()
                pltpu.async_copy(eb, dst, ewsem)

            for half, (ob, sem, ixb, isem) in enumerate(
                    ((ob0, sem0, idx0, isem0), (ob1, sem1, idx1, isem1))):
                c = 2 * t2 + half
                col = pl.multiple_of(c * C, 128)
                pltpu.make_async_copy(
                    ids_hbm.at[pl.ds(col, C)], ixb, isem).wait()

                @pl.when(t2 > 0)
                def _():
                    prev = pl.multiple_of((c - 2) * C, 128)
                    pltpu.make_async_copy(
                        ob, outT_hbm.at[pl.ds(row0, 8), pl.ds(prev, C)],
                        sem).wait()

                @plsc.parallel_loop(0, C // 16)
                def g_body(g):
                    iv = ixb[pl.ds(g * 16, 16)]
                    for k in range(8):
                        vals = plsc.load_gather(
                            tbl.at[pl.ds(k * N_NODES, N_NODES)], [iv])
                        ob[k, pl.ds(g * 16, 16)] = vals

                @pl.when(c + 2 < NCH)
                def _():
                    nxt = pl.multiple_of((c + 2) * C, 128)
                    pltpu.async_copy(ids_hbm.at[pl.ds(nxt, C)], ixb, isem)

                pltpu.async_copy(
                    ob, outT_hbm.at[pl.ds(row0, 8), pl.ds(col, C)], sem)
            return carry

        lax.fori_loop(0, NPAIR, pair, 0)
        last = (NCH - 2) * C
        pltpu.make_async_copy(
            ob0, outT_hbm.at[pl.ds(row0, 8), pl.ds(last, C)], sem0).wait()
        pltpu.make_async_copy(
            ob1, outT_hbm.at[pl.ds(row0, 8), pl.ds(last + C, C)], sem1).wait()
        last_i = wid + NW * (NE_IT - 1)
        _, dlast = e_refs(jnp.where(last_i < NE, last_i, last_i - NW))
        pltpu.make_async_copy(eb, dlast, ewsem).wait()

    @pl.when(t == 0)
    def _():
        v_pass(a_hbm)

    @pl.when(t == 1)
    def _():
        v_pass(b_hbm)


@jax.jit
def _run(vt_flat, eT, ia, ib):
    mesh = plsc.VectorSubcoreMesh(core_axis_name="c", subcore_axis_name="s")
    kern = functools.partial(
        pl.kernel,
        mesh=mesh,
        out_type=jax.ShapeDtypeStruct((D_OUT, N_EDGES), jnp.float32),
        scratch_types=[
            pltpu.VMEM((W_TBL,), jnp.float32),
            pltpu.VMEM((C,), jnp.int32),
            pltpu.VMEM((C,), jnp.int32),
            pltpu.VMEM((8, C), jnp.float32),
            pltpu.VMEM((8, C), jnp.float32),
            pltpu.VMEM((8, C), jnp.float32),
            pltpu.SemaphoreType.DMA,
            pltpu.SemaphoreType.DMA,
            pltpu.SemaphoreType.DMA,
            pltpu.SemaphoreType.DMA,
            pltpu.SemaphoreType.DMA,
            pltpu.SemaphoreType.DMA,
            pltpu.SemaphoreType.DMA,
        ],
        compiler_params=pltpu.CompilerParams(needs_layout_passes=False),
    )(_body)
    return kern(vt_flat, eT, ia, ib)


def kernel(V_set, E_set, a_node_ids, b_node_ids):
    vt_flat = jnp.transpose(V_set[0]).reshape(-1)
    eT = jnp.transpose(E_set[0])
    ia = a_node_ids[0].astype(jnp.int32)
    ib = b_node_ids[0].astype(jnp.int32)
    outT = _run(vt_flat, eT, ia, ib)
    return jnp.transpose(outT)[jnp.newaxis, ...]


# speedup vs baseline: 8.5855x; 1.0009x over previous
import functools

import jax
import jax.numpy as jnp
from jax import lax
from jax.experimental import pallas as pl
from jax.experimental.pallas import tpu as pltpu
from jax.experimental.pallas import tpu_sc as plsc

N_NODES = 10000
N_EDGES = 320000
D_FEAT = 128
D_EDGE = 16
D_OUT = D_EDGE + 2 * D_FEAT

NUM_CORES = 2
NUM_SUBCORES = 16
NW = NUM_CORES * NUM_SUBCORES
C = 1280
NCH = N_EDGES // C
NPAIR = NCH // 2
W_TBL = 8 * N_NODES


def _body(vt_hbm, eT_hbm, a_hbm, b_hbm, outT_hbm, tbl, idx0, idx1, ob0, ob1,
          eb, sem0, sem1, isem0, isem1, tsem, ersem, ewsem):
    wid = lax.axis_index("s") * NUM_CORES + lax.axis_index("c")
    t = wid // NUM_SUBCORES
    rb = wid % NUM_SUBCORES

    pltpu.async_copy(vt_hbm.at[pl.ds(rb * W_TBL, W_TBL)], tbl, tsem)

    NE = 2 * NCH
    NE_IT = (NE + NW - 1) // NW

    def e_refs(i):
        r = pl.multiple_of(jnp.where(i < NCH, 0, 8), 8)
        cc = jnp.where(i < NCH, i, i - NCH)
        col = pl.multiple_of(cc * C, 128)
        return (eT_hbm.at[pl.ds(r, 8), pl.ds(col, C)],
                outT_hbm.at[pl.ds(r, 8), pl.ds(col, C)])

    row0 = D_EDGE + t * D_FEAT + rb * 8

    def v_pass(ids_hbm):
        pltpu.async_copy(ids_hbm.at[pl.ds(0, C)], idx0, isem0)
        pltpu.async_copy(ids_hbm.at[pl.ds(C, C)], idx1, isem1)
        pltpu.make_async_copy(
            vt_hbm.at[pl.ds(rb * W_TBL, W_TBL)], tbl, tsem).wait()

        def pair(t2, carry):
            j = t2 // 2
            i = wid + NW * j

            @pl.when(jnp.logical_and(t2 % 2 == 0, i < NE))
            def _():
                @pl.when(j > 0)
                def _():
                    _, dst_prev = e_refs(i - NW)
                    pltpu.make_async_copy(eb, dst_prev, ewsem).wait()

                src, _ = e_refs(i)
                pltpu.async_copy(src, eb, ersem)

            @pl.when(jnp.logical_and(t2 % 2 == 1, i < NE))
            def _():
                src, dst = e_refs(i)
                pltpu.make_async_copy(src, eb, ersem).wait()
                pltpu.async_copy(eb, dst, ewsem)

            for half, (ob, sem, ixb, isem) in enumerate(
                    ((ob0, sem0, idx0, isem0), (ob1, sem1, idx1, isem1))):
                c = 2 * t2 + half
                col = pl.multiple_of(c * C, 128)
                pltpu.make_async_copy(
                    ids_hbm.at[pl.ds(col, C)], ixb, isem).wait()

                @pl.when(t2 > 0)
                def _():
                    prev = pl.multiple_of((c - 2) * C, 128)
                    pltpu.make_async_copy(
                        ob, outT_hbm.at[pl.ds(row0, 8), pl.ds(prev, C)],
                        sem).wait()

                @plsc.parallel_loop(0, C // 16, unroll=2)
                def g_body(g):
                    iv = ixb[pl.ds(g * 16, 16)]
                    for k in range(8):
                        vals = plsc.load_gather(
                            tbl.at[pl.ds(k * N_NODES, N_NODES)], [iv])
                        ob[k, pl.ds(g * 16, 16)] = vals

                @pl.when(c + 2 < NCH)
                def _():
                    nxt = pl.multiple_of((c + 2) * C, 128)
                    pltpu.async_copy(ids_hbm.at[pl.ds(nxt, C)], ixb, isem)

                pltpu.async_copy(
                    ob, outT_hbm.at[pl.ds(row0, 8), pl.ds(col, C)], sem)
            return carry

        lax.fori_loop(0, NPAIR, pair, 0)
        last = (NCH - 2) * C
        pltpu.make_async_copy(
            ob0, outT_hbm.at[pl.ds(row0, 8), pl.ds(last, C)], sem0).wait()
        pltpu.make_async_copy(
            ob1, outT_hbm.at[pl.ds(row0, 8), pl.ds(last + C, C)], sem1).wait()
        last_i = wid + NW * (NE_IT - 1)
        _, dlast = e_refs(jnp.where(last_i < NE, last_i, last_i - NW))
        pltpu.make_async_copy(eb, dlast, ewsem).wait()

    @pl.when(t == 0)
    def _():
        v_pass(a_hbm)

    @pl.when(t == 1)
    def _():
        v_pass(b_hbm)


@jax.jit
def _run(vt_flat, eT, ia, ib):
    mesh = plsc.VectorSubcoreMesh(core_axis_name="c", subcore_axis_name="s")
    kern = functools.partial(
        pl.kernel,
        mesh=mesh,
        out_type=jax.ShapeDtypeStruct((D_OUT, N_EDGES), jnp.float32),
        scratch_types=[
            pltpu.VMEM((W_TBL,), jnp.float32),
            pltpu.VMEM((C,), jnp.int32),
            pltpu.VMEM((C,), jnp.int32),
            pltpu.VMEM((8, C), jnp.float32),
            pltpu.VMEM((8, C), jnp.float32),
            pltpu.VMEM((8, C), jnp.float32),
            pltpu.SemaphoreType.DMA,
            pltpu.SemaphoreType.DMA,
            pltpu.SemaphoreType.DMA,
            pltpu.SemaphoreType.DMA,
            pltpu.SemaphoreType.DMA,
            pltpu.SemaphoreType.DMA,
            pltpu.SemaphoreType.DMA,
        ],
        compiler_params=pltpu.CompilerParams(needs_layout_passes=False),
    )(_body)
    return kern(vt_flat, eT, ia, ib)


def kernel(V_set, E_set, a_node_ids, b_node_ids):
    vt_flat = jnp.transpose(V_set[0]).reshape(-1)
    eT = jnp.transpose(E_set[0])
    ia = a_node_ids[0].astype(jnp.int32)
    ib = b_node_ids[0].astype(jnp.int32)
    outT = _run(vt_flat, eT, ia, ib)
    return jnp.transpose(outT)[jnp.newaxis, ...]


# speedup vs baseline: 8.7675x; 1.0212x over previous
import functools

import jax
import jax.numpy as jnp
from jax import lax
from jax.experimental import pallas as pl
from jax.experimental.pallas import tpu as pltpu
from jax.experimental.pallas import tpu_sc as plsc

N_NODES = 10000
N_EDGES = 320000
D_FEAT = 128
D_EDGE = 16
D_OUT = D_EDGE + 2 * D_FEAT

NUM_CORES = 2
NUM_SUBCORES = 16
NW = NUM_CORES * NUM_SUBCORES
C = 1280
NCH = N_EDGES // C
NPAIR = NCH // 2
W_TBL = 8 * N_NODES


def _body(vt_hbm, eT_hbm, a_hbm, b_hbm, outT_hbm, tbl, idx0, idx1, ob0, ob1,
          eb, sem0, sem1, isem0, isem1, tsem, ersem, ewsem):
    wid = lax.axis_index("s") * NUM_CORES + lax.axis_index("c")
    t = wid // NUM_SUBCORES
    rb = wid % NUM_SUBCORES

    pltpu.async_copy(vt_hbm.at[pl.ds(rb * 8, 8), :], tbl, tsem)

    NE = 2 * NCH
    NE_IT = (NE + NW - 1) // NW

    def e_refs(i):
        r = pl.multiple_of(jnp.where(i < NCH, 0, 8), 8)
        cc = jnp.where(i < NCH, i, i - NCH)
        col = pl.multiple_of(cc * C, 128)
        return (eT_hbm.at[pl.ds(r, 8), pl.ds(col, C)],
                outT_hbm.at[pl.ds(r, 8), pl.ds(col, C)])

    row0 = D_EDGE + t * D_FEAT + rb * 8

    def v_pass(ids_hbm):
        pltpu.async_copy(ids_hbm.at[pl.ds(0, C)], idx0, isem0)
        pltpu.async_copy(ids_hbm.at[pl.ds(C, C)], idx1, isem1)
        pltpu.make_async_copy(
            vt_hbm.at[pl.ds(rb * 8, 8), :], tbl, tsem).wait()

        def pair(t2, carry):
            j = t2 // 2
            i = wid + NW * j

            @pl.when(jnp.logical_and(t2 % 2 == 0, i < NE))
            def _():
                @pl.when(j > 0)
                def _():
                    _, dst_prev = e_refs(i - NW)
                    pltpu.make_async_copy(eb, dst_prev, ewsem).wait()

                src, _ = e_refs(i)
                pltpu.async_copy(src, eb, ersem)

            @pl.when(jnp.logical_and(t2 % 2 == 1, i < NE))
            def _():
                src, dst = e_refs(i)
                pltpu.make_async_copy(src, eb, ersem).wait()
                pltpu.async_copy(eb, dst, ewsem)

            for half, (ob, sem, ixb, isem) in enumerate(
                    ((ob0, sem0, idx0, isem0), (ob1, sem1, idx1, isem1))):
                c = 2 * t2 + half
                col = pl.multiple_of(c * C, 128)
                pltpu.make_async_copy(
                    ids_hbm.at[pl.ds(col, C)], ixb, isem).wait()

                @pl.when(t2 > 0)
                def _():
                    prev = pl.multiple_of((c - 2) * C, 128)
                    pltpu.make_async_copy(
                        ob, outT_hbm.at[pl.ds(row0, 8), pl.ds(prev, C)],
                        sem).wait()

                @plsc.parallel_loop(0, C // 16, unroll=2)
                def g_body(g):
                    iv = ixb[pl.ds(g * 16, 16)]
                    for k in range(8):
                        vals = plsc.load_gather(
                            tbl, [jnp.full((16,), k, jnp.int32), iv])
                        ob[k, pl.ds(g * 16, 16)] = vals

                @pl.when(c + 2 < NCH)
                def _():
                    nxt = pl.multiple_of((c + 2) * C, 128)
                    pltpu.async_copy(ids_hbm.at[pl.ds(nxt, C)], ixb, isem)

                pltpu.async_copy(
                    ob, outT_hbm.at[pl.ds(row0, 8), pl.ds(col, C)], sem)
            return carry

        lax.fori_loop(0, NPAIR, pair, 0)
        last = (NCH - 2) * C
        pltpu.make_async_copy(
            ob0, outT_hbm.at[pl.ds(row0, 8), pl.ds(last, C)], sem0).wait()
        pltpu.make_async_copy(
            ob1, outT_hbm.at[pl.ds(row0, 8), pl.ds(last + C, C)], sem1).wait()
        last_i = wid + NW * (NE_IT - 1)
        _, dlast = e_refs(jnp.where(last_i < NE, last_i, last_i - NW))
        pltpu.make_async_copy(eb, dlast, ewsem).wait()

    @pl.when(t == 0)
    def _():
        v_pass(a_hbm)

    @pl.when(t == 1)
    def _():
        v_pass(b_hbm)


@jax.jit
def _run(vt_flat, eT, ia, ib):
    mesh = plsc.VectorSubcoreMesh(core_axis_name="c", subcore_axis_name="s")
    kern = functools.partial(
        pl.kernel,
        mesh=mesh,
        out_type=jax.ShapeDtypeStruct((D_OUT, N_EDGES), jnp.float32),
        scratch_types=[
            pltpu.VMEM((8, N_NODES), jnp.float32),
            pltpu.VMEM((C,), jnp.int32),
            pltpu.VMEM((C,), jnp.int32),
            pltpu.VMEM((8, C), jnp.float32),
            pltpu.VMEM((8, C), jnp.float32),
            pltpu.VMEM((8, C), jnp.float32),
            pltpu.SemaphoreType.DMA,
            pltpu.SemaphoreType.DMA,
            pltpu.SemaphoreType.DMA,
            pltpu.SemaphoreType.DMA,
            pltpu.SemaphoreType.DMA,
            pltpu.SemaphoreType.DMA,
            pltpu.SemaphoreType.DMA,
        ],
        compiler_params=pltpu.CompilerParams(needs_layout_passes=False),
    )(_body)
    return kern(vt_flat, eT, ia, ib)


def kernel(V_set, E_set, a_node_ids, b_node_ids):
    vt = jnp.transpose(V_set[0])
    eT = jnp.transpose(E_set[0])
    ia = a_node_ids[0].astype(jnp.int32)
    ib = b_node_ids[0].astype(jnp.int32)
    outT = _run(vt, eT, ia, ib)
    return jnp.transpose(outT)[jnp.newaxis, ...]


# speedup vs baseline: 9.7648x; 1.1137x over previous
import functools

import jax
import jax.numpy as jnp
from jax import lax
from jax.experimental import pallas as pl
from jax.experimental.pallas import tpu as pltpu
from jax.experimental.pallas import tpu_sc as plsc

N_NODES = 10000
N_EDGES = 320000
D_FEAT = 128
D_EDGE = 16
D_OUT = D_EDGE + 2 * D_FEAT

NUM_CORES = 2
NUM_SUBCORES = 16
NW = NUM_CORES * NUM_SUBCORES
C = 1280
NCH = N_EDGES // C
NPAIR = NCH // 2
W_TBL = 8 * N_NODES


def _body(vt_hbm, eT_hbm, a_hbm, b_hbm, outT_hbm, tbl, idx0, idx1, ob0, ob1,
          eb, sem0, sem1, isem0, isem1, tsem, ersem, ewsem):
    wid = lax.axis_index("s") * NUM_CORES + lax.axis_index("c")
    t = wid // NUM_SUBCORES
    rb = wid % NUM_SUBCORES

    pltpu.async_copy(vt_hbm.at[pl.ds(rb * 8, 8), :], tbl, tsem)

    NE = 2 * NCH
    NE_IT = (NE + NW - 1) // NW

    def e_refs(i):
        r = pl.multiple_of(jnp.where(i < NCH, 0, 8), 8)
        cc = jnp.where(i < NCH, i, i - NCH)
        col = pl.multiple_of(cc * C, 128)
        return (eT_hbm.at[pl.ds(r, 8), pl.ds(col, C)],
                outT_hbm.at[pl.ds(r, 8), pl.ds(col, C)])

    row0 = D_EDGE + t * D_FEAT + rb * 8

    def v_pass(ids_hbm):
        pltpu.async_copy(ids_hbm.at[pl.ds(0, 2 * C)], idx0, isem0)
        pltpu.async_copy(ids_hbm.at[pl.ds(2 * C, 2 * C)], idx1, isem1)
        pltpu.make_async_copy(
            vt_hbm.at[pl.ds(rb * 8, 8), :], tbl, tsem).wait()

        def pair(t2, carry):
            j = t2 // 2
            i = wid + NW * j

            @pl.when(jnp.logical_and(t2 % 2 == 0, i < NE))
            def _():
                @pl.when(j > 0)
                def _():
                    _, dst_prev = e_refs(i - NW)
                    pltpu.make_async_copy(eb, dst_prev, ewsem).wait()

                src, _ = e_refs(i)
                pltpu.async_copy(src, eb, ersem)

            @pl.when(jnp.logical_and(t2 % 2 == 1, i < NE))
            def _():
                src, dst = e_refs(i)
                pltpu.make_async_copy(src, eb, ersem).wait()
                pltpu.async_copy(eb, dst, ewsem)

            pidx = t2 % 2

            @pl.when(pidx == 0)
            def _():
                pltpu.make_async_copy(
                    ids_hbm.at[pl.ds(0, 2 * C)], idx0, isem0).wait()

            @pl.when(pidx == 1)
            def _():
                pltpu.make_async_copy(
                    ids_hbm.at[pl.ds(0, 2 * C)], idx1, isem1).wait()

            for half, (ob, sem) in enumerate(((ob0, sem0), (ob1, sem1))):
                c = 2 * t2 + half
                col = pl.multiple_of(c * C, 128)

                @pl.when(t2 > 0)
                def _():
                    prev = pl.multiple_of((c - 2) * C, 128)
                    pltpu.make_async_copy(
                        ob, outT_hbm.at[pl.ds(row0, 8), pl.ds(prev, C)],
                        sem).wait()

                for p, ixb in enumerate((idx0, idx1)):
                    @pl.when(pidx == p)
                    def _():
                        @plsc.parallel_loop(0, C // 16, unroll=2)
                        def g_body(g):
                            iv = ixb[pl.ds(half * C + g * 16, 16)]
                            for k in range(8):
                                vals = plsc.load_gather(
                                    tbl, [jnp.full((16,), k, jnp.int32), iv])
                                ob[k, pl.ds(g * 16, 16)] = vals

                pltpu.async_copy(
                    ob, outT_hbm.at[pl.ds(row0, 8), pl.ds(col, C)], sem)

            @pl.when(t2 + 2 < NPAIR)
            def _():
                nxt = pl.multiple_of((t2 + 2) * 2 * C, 128)

                @pl.when(pidx == 0)
                def _():
                    pltpu.async_copy(
                        ids_hbm.at[pl.ds(nxt, 2 * C)], idx0, isem0)

                @pl.when(pidx == 1)
                def _():
                    pltpu.async_copy(
                        ids_hbm.at[pl.ds(nxt, 2 * C)], idx1, isem1)

            return carry

        lax.fori_loop(0, NPAIR, pair, 0)
        last = (NCH - 2) * C
        pltpu.make_async_copy(
            ob0, outT_hbm.at[pl.ds(row0, 8), pl.ds(last, C)], sem0).wait()
        pltpu.make_async_copy(
            ob1, outT_hbm.at[pl.ds(row0, 8), pl.ds(last + C, C)], sem1).wait()
        last_i = wid + NW * (NE_IT - 1)
        _, dlast = e_refs(jnp.where(last_i < NE, last_i, last_i - NW))
        pltpu.make_async_copy(eb, dlast, ewsem).wait()

    @pl.when(t == 0)
    def _():
        v_pass(a_hbm)

    @pl.when(t == 1)
    def _():
        v_pass(b_hbm)


@jax.jit
def _run(vt_flat, eT, ia, ib):
    mesh = plsc.VectorSubcoreMesh(core_axis_name="c", subcore_axis_name="s")
    kern = functools.partial(
        pl.kernel,
        mesh=mesh,
        out_type=jax.ShapeDtypeStruct((D_OUT, N_EDGES), jnp.float32),
        scratch_types=[
            pltpu.VMEM((8, N_NODES), jnp.float32),
            pltpu.VMEM((2 * C,), jnp.int32),
            pltpu.VMEM((2 * C,), jnp.int32),
            pltpu.VMEM((8, C), jnp.float32),
            pltpu.VMEM((8, C), jnp.float32),
            pltpu.VMEM((8, C), jnp.float32),
            pltpu.SemaphoreType.DMA,
            pltpu.SemaphoreType.DMA,
            pltpu.SemaphoreType.DMA,
            pltpu.SemaphoreType.DMA,
            pltpu.SemaphoreType.DMA,
            pltpu.SemaphoreType.DMA,
            pltpu.SemaphoreType.DMA,
        ],
        compiler_params=pltpu.CompilerParams(needs_layout_passes=False),
    )(_body)
    return kern(vt_flat, eT, ia, ib)


def kernel(V_set, E_set, a_node_ids, b_node_ids):
    vt = jnp.transpose(V_set[0])
    eT = jnp.transpose(E_set[0])
    ia = a_node_ids[0].astype(jnp.int32)
    ib = b_node_ids[0].astype(jnp.int32)
    outT = _run(vt, eT, ia, ib)
    return jnp.transpose(outT)[jnp.newaxis, ...]


# speedup vs baseline: 9.7673x; 1.0003x over previous
import functools

import jax
import jax.numpy as jnp
from jax import lax
from jax.experimental import pallas as pl
from jax.experimental.pallas import tpu as pltpu
from jax.experimental.pallas import tpu_sc as plsc

N_NODES = 10000
N_EDGES = 320000
D_FEAT = 128
D_EDGE = 16
D_OUT = D_EDGE + 2 * D_FEAT

NUM_CORES = 2
NUM_SUBCORES = 16
NW = NUM_CORES * NUM_SUBCORES
C = 1280
NCH = N_EDGES // C
NPAIR = NCH // 2


def _body(vt_hbm, eT_hbm, a_hbm, b_hbm, outT_hbm, tbl, idx0, idx1, ob0, ob1,
          eb, sem0, sem1, isem0, isem1, tsem, ersem, ewsem):
    wid = lax.axis_index("s") * NUM_CORES + lax.axis_index("c")
    t = wid // NUM_SUBCORES
    rb = wid % NUM_SUBCORES

    pltpu.async_copy(vt_hbm.at[pl.ds(rb * 8, 8), :], tbl, tsem)

    NE = 2 * NCH
    NE_IT = (NE + NW - 1) // NW

    def e_refs(i):
        r = pl.multiple_of(jnp.where(i < NCH, 0, 8), 8)
        cc = jnp.where(i < NCH, i, i - NCH)
        col = pl.multiple_of(cc * C, 128)
        return (eT_hbm.at[pl.ds(r, 8), pl.ds(col, C)],
                outT_hbm.at[pl.ds(r, 8), pl.ds(col, C)])

    row0 = D_EDGE + t * D_FEAT + rb * 8

    def v_pass(ids_hbm):
        pltpu.async_copy(ids_hbm.at[pl.ds(0, 2 * C)], idx0, isem0)
        pltpu.async_copy(ids_hbm.at[pl.ds(2 * C, 2 * C)], idx1, isem1)
        pltpu.make_async_copy(
            vt_hbm.at[pl.ds(rb * 8, 8), :], tbl, tsem).wait()

        def pair(t2, carry):
            j = t2 // 2
            i = wid + NW * j

            @pl.when(jnp.logical_and(t2 % 2 == 0, i < NE))
            def _():
                @pl.when(j > 0)
                def _():
                    _, dst_prev = e_refs(i - NW)
                    pltpu.make_async_copy(eb, dst_prev, ewsem).wait()

                src, _ = e_refs(i)
                pltpu.async_copy(src, eb, ersem)

            @pl.when(jnp.logical_and(t2 % 2 == 1, i < NE))
            def _():
                src, dst = e_refs(i)
                pltpu.make_async_copy(src, eb, ersem).wait()
                pltpu.async_copy(eb, dst, ewsem)

            pidx = t2 % 2

            @pl.when(pidx == 0)
            def _():
                pltpu.make_async_copy(
                    ids_hbm.at[pl.ds(0, 2 * C)], idx0, isem0).wait()

            @pl.when(pidx == 1)
            def _():
                pltpu.make_async_copy(
                    ids_hbm.at[pl.ds(0, 2 * C)], idx1, isem1).wait()

            for half, (ob, sem) in enumerate(((ob0, sem0), (ob1, sem1))):
                c = 2 * t2 + half
                col = pl.multiple_of(c * C, 128)

                @pl.when(t2 > 0)
                def _():
                    prev = pl.multiple_of((c - 2) * C, 128)
                    pltpu.make_async_copy(
                        ob, outT_hbm.at[pl.ds(row0, 8), pl.ds(prev, C)],
                        sem).wait()

                for p, ixb in enumerate((idx0, idx1)):
                    @pl.when(pidx == p)
                    def _():
                        @plsc.parallel_loop(0, C // 16, unroll=2)
                        def g_body(g):
                            iv = ixb[pl.ds(half * C + g * 16, 16)]
                            for k in range(8):
                                vals = plsc.load_gather(
                                    tbl, [jnp.full((16,), k, jnp.int32), iv])
                                ob[k, pl.ds(g * 16, 16)] = vals

                pltpu.async_copy(
                    ob, outT_hbm.at[pl.ds(row0, 8), pl.ds(col, C)], sem)

            @pl.when(t2 + 2 < NPAIR)
            def _():
                nxt = pl.multiple_of((t2 + 2) * 2 * C, 128)

                @pl.when(pidx == 0)
                def _():
                    pltpu.async_copy(
                        ids_hbm.at[pl.ds(nxt, 2 * C)], idx0, isem0)

                @pl.when(pidx == 1)
                def _():
                    pltpu.async_copy(
                        ids_hbm.at[pl.ds(nxt, 2 * C)], idx1, isem1)

            return carry

        lax.fori_loop(0, NPAIR, pair, 0)
        last = (NCH - 2) * C
        pltpu.make_async_copy(
            ob0, outT_hbm.at[pl.ds(row0, 8), pl.ds(last, C)], sem0).wait()
        pltpu.make_async_copy(
            ob1, outT_hbm.at[pl.ds(row0, 8), pl.ds(last + C, C)], sem1).wait()
        last_i = wid + NW * (NE_IT - 1)
        _, dlast = e_refs(jnp.where(last_i < NE, last_i, last_i - NW))
        pltpu.make_async_copy(eb, dlast, ewsem).wait()

    @pl.when(t == 0)
    def _():
        v_pass(a_hbm)

    @pl.when(t == 1)
    def _():
        v_pass(b_hbm)


@jax.jit
def _run(vt_flat, eT, ia, ib):
    mesh = plsc.VectorSubcoreMesh(core_axis_name="c", subcore_axis_name="s")
    kern = functools.partial(
        pl.kernel,
        mesh=mesh,
        out_type=jax.ShapeDtypeStruct((D_OUT, N_EDGES), jnp.float32),
        scratch_types=[
            pltpu.VMEM((8, N_NODES), jnp.float32),
            pltpu.VMEM((2 * C,), jnp.int32),
            pltpu.VMEM((2 * C,), jnp.int32),
            pltpu.VMEM((8, C), jnp.float32),
            pltpu.VMEM((8, C), jnp.float32),
            pltpu.VMEM((8, C), jnp.float32),
            pltpu.SemaphoreType.DMA,
            pltpu.SemaphoreType.DMA,
            pltpu.SemaphoreType.DMA,
            pltpu.SemaphoreType.DMA,
            pltpu.SemaphoreType.DMA,
            pltpu.SemaphoreType.DMA,
            pltpu.SemaphoreType.DMA,
        ],
        compiler_params=pltpu.CompilerParams(needs_layout_passes=False),
    )(_body)
    return kern(vt_flat, eT, ia, ib)


def kernel(V_set, E_set, a_node_ids, b_node_ids):
    vt = jnp.transpose(V_set[0])
    eT = jnp.transpose(E_set[0])
    ia = a_node_ids[0].astype(jnp.int32)
    ib = b_node_ids[0].astype(jnp.int32)
    outT = _run(vt, eT, ia, ib)
    return jnp.transpose(outT)[jnp.newaxis, ...]
